# baseline (speedup 1.0000x reference)
import functools

import jax
import jax.numpy as jnp
from jax import lax
from jax.experimental import pallas as pl
from jax.experimental.pallas import tpu as pltpu
from jax.experimental.pallas import tpu_sc as plsc

_ROWS = 6
_LANES = 16


@functools.lru_cache(maxsize=1)
def _build_row_dot():
    mesh = plsc.VectorSubcoreMesh(core_axis_name="c", subcore_axis_name="s")

    @functools.partial(
        pl.kernel,
        out_type=jax.ShapeDtypeStruct((_LANES,), jnp.float32),
        mesh=mesh,
        scratch_types=[
            pltpu.VMEM((_ROWS * _LANES,), jnp.float32),
            pltpu.VMEM((_LANES,), jnp.float32),
            pltpu.VMEM((_LANES,), jnp.int32),
            pltpu.VMEM((_LANES,), jnp.float32),
        ],
    )
    def _row_dot(m_hbm, t_hbm, s_hbm, out_hbm, m_v, t_v, s_v, o_v):
        is_lead = (lax.axis_index("c") == 0) & (lax.axis_index("s") == 0)

        @pl.when(is_lead)
        def _():
            pltpu.sync_copy(m_hbm, m_v)
            pltpu.sync_copy(t_hbm, t_v)
            pltpu.sync_copy(s_hbm, s_v)

        sel = s_v[...]
        row = jnp.zeros((_LANES,), jnp.float32)
        for i in range(_ROWS):
            r = m_v[pl.ds(i * _LANES, _LANES)]
            row = jnp.where(sel == i, r, row)
        p = row * t_v[...]
        lanes = lax.iota(jnp.int32, _LANES)
        for sh in (8, 4, 2, 1):
            p = p + p.at[(lanes + sh) & (_LANES - 1)].get(mode="promise_in_bounds")
        o_v[...] = p

        @pl.when(is_lead)
        def _():
            pltpu.sync_copy(o_v, out_hbm)

    return _row_dot


def kernel(M, theta, state):
    m_pad = jnp.zeros((_ROWS, _LANES), jnp.float32)
    m_pad = m_pad.at[:, : M.shape[1]].set(M).reshape(_ROWS * _LANES)
    t_pad = jnp.zeros((_LANES,), jnp.float32).at[: theta.shape[0]].set(theta)
    s_vec = jnp.full((_LANES,), state, jnp.int32)
    out = _build_row_dot()(m_pad, t_pad, s_vec)
    return out[0]


# speedup vs baseline: 1.0846x; 1.0846x over previous
import functools

import jax
import jax.numpy as jnp
from jax import lax
from jax.experimental import pallas as pl
from jax.experimental.pallas import tpu as pltpu
from jax.experimental.pallas import tpu_sc as plsc

_ROWS = 6
_LANES = 16


@functools.lru_cache(maxsize=1)
def _build_row_dot():
    mesh = plsc.VectorSubcoreMesh(
        core_axis_name="c", subcore_axis_name="s", num_cores=1, num_subcores=1
    )

    @functools.partial(
        pl.kernel,
        out_type=jax.ShapeDtypeStruct((_LANES,), jnp.float32),
        mesh=mesh,
        scratch_types=[
            pltpu.VMEM((_ROWS * _LANES,), jnp.float32),
            pltpu.VMEM((_LANES,), jnp.float32),
            pltpu.VMEM((_LANES,), jnp.int32),
            pltpu.VMEM((_LANES,), jnp.float32),
        ],
    )
    def _row_dot(m_hbm, t_hbm, s_hbm, out_hbm, m_v, t_v, s_v, o_v):
        pltpu.sync_copy(m_hbm, m_v)
        pltpu.sync_copy(t_hbm, t_v)
        pltpu.sync_copy(s_hbm, s_v)

        sel = s_v[...]
        row = jnp.zeros((_LANES,), jnp.float32)
        for i in range(_ROWS):
            r = m_v[pl.ds(i * _LANES, _LANES)]
            row = jnp.where(sel == i, r, row)
        p = row * t_v[...]
        lanes = lax.iota(jnp.int32, _LANES)
        for sh in (8, 4, 2, 1):
            p = p + p.at[(lanes + sh) & (_LANES - 1)].get(mode="promise_in_bounds")
        o_v[...] = p
        pltpu.sync_copy(o_v, out_hbm)

    return _row_dot


def kernel(M, theta, state):
    m_pad = jnp.zeros((_ROWS, _LANES), jnp.float32)
    m_pad = m_pad.at[:, : M.shape[1]].set(M).reshape(_ROWS * _LANES)
    t_pad = jnp.zeros((_LANES,), jnp.float32).at[: theta.shape[0]].set(theta)
    s_vec = jnp.full((_LANES,), state, jnp.int32)
    out = _build_row_dot()(m_pad, t_pad, s_vec)
    return out[0]


# speedup vs baseline: 10.7991x; 9.9563x over previous
import jax
import jax.numpy as jnp
from jax.experimental import pallas as pl
from jax.experimental.pallas import tpu as pltpu


def _row_dot(s_ref, m_ref, t_ref, o_ref):
    i = s_ref[0]
    row = m_ref[pl.ds(i, 1), :]
    o_ref[0, 0] = jnp.sum(row * t_ref[...])


def kernel(M, theta, state):
    s = jnp.asarray(state, jnp.int32).reshape(1)
    t2 = theta.reshape(1, theta.shape[0])
    out = pl.pallas_call(
        _row_dot,
        out_shape=jax.ShapeDtypeStruct((1, 1), jnp.float32),
        in_specs=[
            pl.BlockSpec(memory_space=pltpu.SMEM),
            pl.BlockSpec(memory_space=pltpu.VMEM),
            pl.BlockSpec(memory_space=pltpu.VMEM),
        ],
        out_specs=pl.BlockSpec(memory_space=pltpu.SMEM),
    )(s, M, t2)
    return out.reshape(())
